# baseline (speedup 1.0000x reference)
import functools
import jax
import jax.numpy as jnp
from jax import lax
from jax.experimental import pallas as pl
from jax.experimental.pallas import tpu as pltpu
from jax.experimental.pallas import tpu_sc as plsc

D = 768
E = 8
T = 64
TB = 4
NC = 2
NS = 16
NW = NC * NS
L = 16
NK = D // L


def _make_sc_router(m_sc):
    tok_w = m_sc // NS
    nsub = tok_w // T
    mesh = plsc.VectorSubcoreMesh(core_axis_name="c", subcore_axis_name="s",
                                  num_cores=1)

    @functools.partial(
        pl.kernel,
        out_type=jax.ShapeDtypeStruct((m_sc * E,), jnp.float32),
        mesh=mesh,
        scratch_types=[
            pltpu.VMEM((2, T * D), jnp.float32),
            pltpu.VMEM((D * E,), jnp.float32),
            pltpu.VMEM((2, T * E), jnp.float32),
            pltpu.VMEM((2 * L * L,), jnp.float32),
            pltpu.SemaphoreType.DMA((2,)),
            pltpu.SemaphoreType.DMA((2,)),
        ],
        compiler_params=pltpu.CompilerParams(
            use_tc_tiling_on_sc=False, needs_layout_passes=False),
    )
    def sc_router(x_hbm, wt_hbm, o_hbm, xbuf, wv, obuf, red, isems, osems):
        wid = lax.axis_index("s")
        base = wid * tok_w
        pltpu.sync_copy(wt_hbm, wv)
        iota = lax.iota(jnp.int32, L)

        def icopy(j):
            return pltpu.make_async_copy(
                x_hbm.at[pl.ds((base + j * T) * D, T * D)],
                xbuf.at[j % 2],
                isems.at[j % 2],
            )

        def ocopy(j):
            return pltpu.make_async_copy(
                obuf.at[j % 2],
                o_hbm.at[pl.ds((base + j * T) * E, T * E)],
                osems.at[j % 2],
            )

        icopy(0).start()
        for j in range(nsub):
            if j + 1 < nsub:
                icopy(j + 1).start()
            icopy(j).wait()
            if j >= 2:
                ocopy(j - 2).wait()
            xb = xbuf.at[j % 2]
            ob = obuf.at[j % 2]

            def tb_body(tb, _):
                def k_body(k, accs):
                    koff = pl.multiple_of(k * L, L)
                    ws = [wv[pl.ds(pl.multiple_of(e * D + k * L, L), L)]
                          for e in range(E)]
                    new = []
                    for c in range(TB):
                        toff = pl.multiple_of((tb * TB + c) * D + k * L, L)
                        xv = xb[pl.ds(toff, L)]
                        new.append(tuple(accs[c][e] + xv * ws[e]
                                         for e in range(E)))
                    return tuple(new)

                zero = jnp.zeros((L,), jnp.float32)
                init = tuple(tuple(zero for _ in range(E)) for _ in range(TB))
                accs = lax.fori_loop(0, NK, k_body, init, unroll=4)
                for c in range(TB):
                    for e in range(E):
                        r = c * E + e
                        red[pl.ds(r * L, L)] = accs[c][e]
                for blk in range(2):
                    tot = None
                    for l in range(L):
                        idx = blk * L * L + iota * L + ((l + iota) & (L - 1))
                        dv = plsc.load_gather(red, [idx])
                        tot = dv if tot is None else tot + dv
                    ooff = pl.multiple_of((tb * TB) * E + blk * L, L)
                    ob[pl.ds(ooff, L)] = tot
                return 0

            lax.fori_loop(0, T // TB, tb_body, 0)
            ocopy(j).start()
        for j in range(max(nsub - 2, 0), nsub):
            ocopy(j).wait()

    return sc_router


M_SC = 8192


def kernel(x, W):
    B, S, D_ = x.shape
    M = B * S
    x2 = x.reshape(M, D_)
    wt = W.T.reshape(D * E)
    h = M_SC // 2
    sc = _make_sc_router(h)
    out_sc0 = sc(x2[:h].reshape(h * D), wt).reshape(h, E)
    out_sc1 = sc(x2[h:M_SC].reshape(h * D), wt).reshape(h, E)
    out_tc = jnp.dot(x2[M_SC:], W)
    out = jnp.concatenate([out_sc0, out_sc1, out_tc], axis=0)
    return out.reshape(B, S, E)


# speedup vs baseline: 4.2798x; 4.2798x over previous
import functools
import jax
import jax.numpy as jnp
from jax import lax
from jax.experimental import pallas as pl
from jax.experimental.pallas import tpu as pltpu
from jax.experimental.pallas import tpu_sc as plsc

D = 768
E = 8
T = 64
TB = 4
NC = 2
NS = 16
NW = NC * NS
L = 16
NK = D // L


def _make_sc_router(m_sc):
    tok_w = m_sc // NW
    nsub = tok_w // T
    mesh = plsc.VectorSubcoreMesh(core_axis_name="c", subcore_axis_name="s")

    @functools.partial(
        pl.kernel,
        out_type=jax.ShapeDtypeStruct((m_sc * E,), jnp.float32),
        mesh=mesh,
        scratch_types=[
            pltpu.VMEM((2, T * D), jnp.float32),
            pltpu.VMEM((D * E,), jnp.float32),
            pltpu.VMEM((2, T * E), jnp.float32),
            pltpu.VMEM((2 * L * L,), jnp.float32),
            pltpu.SemaphoreType.DMA((2,)),
            pltpu.SemaphoreType.DMA((2,)),
        ],
        compiler_params=pltpu.CompilerParams(
            use_tc_tiling_on_sc=False, needs_layout_passes=False),
    )
    def sc_router(x_hbm, wt_hbm, o_hbm, xbuf, wv, obuf, red, isems, osems):
        wid = lax.axis_index("s") * NC + lax.axis_index("c")
        base = wid * tok_w
        pltpu.sync_copy(wt_hbm, wv)
        iota = lax.iota(jnp.int32, L)

        def icopy(j):
            return pltpu.make_async_copy(
                x_hbm.at[pl.ds((base + j * T) * D, T * D)],
                xbuf.at[j % 2],
                isems.at[j % 2],
            )

        def ocopy(j):
            return pltpu.make_async_copy(
                obuf.at[j % 2],
                o_hbm.at[pl.ds((base + j * T) * E, T * E)],
                osems.at[j % 2],
            )

        icopy(0).start()
        for j in range(nsub):
            if j + 1 < nsub:
                icopy(j + 1).start()
            icopy(j).wait()
            if j >= 2:
                ocopy(j - 2).wait()
            xb = xbuf.at[j % 2]
            ob = obuf.at[j % 2]

            def tb_body(tb, _):
                def k_body(k, accs):
                    koff = pl.multiple_of(k * L, L)
                    ws = [wv[pl.ds(pl.multiple_of(e * D + k * L, L), L)]
                          for e in range(E)]
                    new = []
                    for c in range(TB):
                        toff = pl.multiple_of((tb * TB + c) * D + k * L, L)
                        xv = xb[pl.ds(toff, L)]
                        new.append(tuple(accs[c][e] + xv * ws[e]
                                         for e in range(E)))
                    return tuple(new)

                zero = jnp.zeros((L,), jnp.float32)
                init = tuple(tuple(zero for _ in range(E)) for _ in range(TB))
                accs = lax.fori_loop(0, NK, k_body, init)
                for c in range(TB):
                    for e in range(E):
                        r = c * E + e
                        red[pl.ds(r * L, L)] = accs[c][e]
                for blk in range(2):
                    tot = None
                    for l in range(L):
                        idx = blk * L * L + iota * L + ((l + iota) & (L - 1))
                        dv = plsc.load_gather(red, [idx])
                        tot = dv if tot is None else tot + dv
                    ooff = pl.multiple_of((tb * TB) * E + blk * L, L)
                    ob[pl.ds(ooff, L)] = tot
                return 0

            lax.fori_loop(0, T // TB, tb_body, 0)
            ocopy(j).start()
        for j in range(max(nsub - 2, 0), nsub):
            ocopy(j).wait()

    return sc_router


M_SC = 2048


def kernel(x, W):
    B, S, D_ = x.shape
    M = B * S
    x2 = x.reshape(M, D_)
    wt = W.T.reshape(D * E)
    out_sc = _make_sc_router(M_SC)(
        x2[:M_SC].reshape(M_SC * D), wt).reshape(M_SC, E)
    out_tc = jnp.dot(x2[M_SC:], W)
    out = jnp.concatenate([out_sc, out_tc], axis=0)
    return out.reshape(B, S, E)


# speedup vs baseline: 5.3628x; 1.2530x over previous
import jax
import jax.numpy as jnp
from jax.experimental import pallas as pl
from jax.experimental.pallas import tpu as pltpu

D = 768
E = 8
CHUNKS = (7680, 7680, 7680, 7680, 2048)
CMAX = 7680
NBUF = 2


def _router_body(x_hbm, w_ref, o_hbm, xbuf, obuf, isems, osems):
    nchunks = len(CHUNKS)
    offs = [sum(CHUNKS[:i]) for i in range(nchunks)]

    def icopy(i):
        return pltpu.make_async_copy(
            x_hbm.at[pl.ds(offs[i], CHUNKS[i]), :],
            xbuf.at[i % NBUF, pl.ds(0, CHUNKS[i]), :],
            isems.at[i % NBUF],
        )

    def ocopy(i):
        return pltpu.make_async_copy(
            obuf.at[i % NBUF, pl.ds(0, CHUNKS[i]), :],
            o_hbm.at[pl.ds(offs[i], CHUNKS[i]), :],
            osems.at[i % NBUF],
        )

    for i in range(min(NBUF, nchunks)):
        icopy(i).start()
    for i in range(nchunks):
        icopy(i).wait()
        if i >= NBUF:
            ocopy(i - NBUF).wait()
        obuf[i % NBUF, pl.ds(0, CHUNKS[i]), :] = jnp.dot(
            xbuf[i % NBUF, pl.ds(0, CHUNKS[i]), :], w_ref[...],
            preferred_element_type=jnp.float32)
        ocopy(i).start()
        if i + NBUF < nchunks:
            icopy(i + NBUF).start()
    for i in range(max(nchunks - NBUF, 0), nchunks):
        ocopy(i).wait()


def kernel(x, W):
    B, S, D_ = x.shape
    M = B * S
    x2 = x.reshape(M, D_)
    out = pl.pallas_call(
        _router_body,
        in_specs=[
            pl.BlockSpec(memory_space=pltpu.MemorySpace.HBM),
            pl.BlockSpec(memory_space=pltpu.MemorySpace.VMEM),
        ],
        out_specs=pl.BlockSpec(memory_space=pltpu.MemorySpace.HBM),
        out_shape=jax.ShapeDtypeStruct((M, E), jnp.float32),
        scratch_shapes=[
            pltpu.VMEM((NBUF, CMAX, D), jnp.float32),
            pltpu.VMEM((NBUF, CMAX, E), jnp.float32),
            pltpu.SemaphoreType.DMA((NBUF,)),
            pltpu.SemaphoreType.DMA((NBUF,)),
        ],
    )(x2, W)
    return out.reshape(B, S, E)


# speedup vs baseline: 5.7765x; 1.0771x over previous
import jax
import jax.numpy as jnp
from jax.experimental import pallas as pl

BLK = 4096


def _router_body(x_ref, w_ref, o_ref):
    o_ref[...] = jnp.dot(x_ref[...], w_ref[...],
                         preferred_element_type=jnp.float32)


def kernel(x, W):
    B, S, D = x.shape
    E = W.shape[1]
    M = B * S
    x2 = x.reshape(M, D)
    out = pl.pallas_call(
        _router_body,
        grid=(M // BLK,),
        in_specs=[
            pl.BlockSpec((BLK, D), lambda i: (i, 0)),
            pl.BlockSpec((D, E), lambda i: (0, 0)),
        ],
        out_specs=pl.BlockSpec((BLK, E), lambda i: (i, 0)),
        out_shape=jax.ShapeDtypeStruct((M, E), jnp.float32),
    )(x2, W)
    return out.reshape(B, S, E)
